# baseline (speedup 1.0000x reference)
import jax
import jax.numpy as jnp
from jax import lax
from jax.experimental import pallas as pl
from jax.experimental.pallas import tpu as pltpu
from jax.experimental.pallas import tpu_sc as plsc

_ROWS = 16384
_COLS = 200
_N = _ROWS * _COLS
_NB = 25
_CHUNK = 16384
_NCHUNK = 200
_NW = 32
_MAXK = 7


def _gather16(w16, iv):
    return lax.gather(
        w16, iv[:, None],
        lax.GatherDimensionNumbers(
            offset_dims=(), collapsed_slice_dims=(0,), start_index_map=(0,)),
        slice_sizes=(1,),
        mode=lax.GatherScatterMode.PROMISE_IN_BOUNDS)


def _sc_body(x_hbm, w_hbm, out_hbm, w_v, idx0, idx1, res0, res1,
             isem0, isem1, osem0, osem1):
    wid = lax.axis_index("s") * 2 + lax.axis_index("c")
    pltpu.sync_copy(w_hbm, w_v)
    w16 = w_v[...]

    idx = (idx0, idx1)
    res = (res0, res1)
    isem = (isem0, isem1)
    osem = (osem0, osem1)

    def chunk_id(i):
        return wid + i * _NW

    def in_dma(i, buf):
        k = chunk_id(i)
        return pltpu.make_async_copy(
            x_hbm.at[k // 8, k % 8, :], idx[buf], isem[buf])

    def out_dma(i, buf):
        k = chunk_id(i)
        return pltpu.make_async_copy(
            res[buf], out_hbm.at[pl.ds(k * _CHUNK, _CHUNK)], osem[buf])

    @pl.when(chunk_id(0) < _NCHUNK)
    def _():
        in_dma(0, 0).start()

    for i in range(_MAXK):
        buf = i & 1

        @pl.when(chunk_id(i) < _NCHUNK)
        def _(i=i, buf=buf):
            in_dma(i, buf).wait()
            if i + 1 < _MAXK:
                @pl.when(chunk_id(i + 1) < _NCHUNK)
                def _():
                    in_dma(i + 1, 1 - buf).start()
            if i >= 2:
                out_dma(i - 2, buf).wait()

            def body(j, _):
                for u in range(8):
                    p = j * 128 + u * 16
                    res[buf][pl.ds(p, 16)] = _gather16(
                        w16, idx[buf][pl.ds(p, 16)])
                return 0

            lax.fori_loop(0, 128, body, 0)
            out_dma(i, buf).start()

    for i in (_MAXK - 2, _MAXK - 1):
        @pl.when(chunk_id(i) < _NCHUNK)
        def _(i=i):
            out_dma(i, i & 1).wait()


@jax.jit
def kernel(x, weight):
    x3 = x.T.reshape(_NB, 8, _ROWS)
    w_flat = jnp.pad(weight.reshape(-1).astype(jnp.float32), (0, 12))
    mesh = plsc.VectorSubcoreMesh(core_axis_name="c", subcore_axis_name="s")
    out_lin = pl.kernel(
        _sc_body,
        mesh=mesh,
        out_type=jax.ShapeDtypeStruct((_N,), jnp.float32),
        scratch_types=[
            pltpu.VMEM((16,), jnp.float32),
            pltpu.VMEM((_CHUNK,), jnp.int32),
            pltpu.VMEM((_CHUNK,), jnp.int32),
            pltpu.VMEM((_CHUNK,), jnp.float32),
            pltpu.VMEM((_CHUNK,), jnp.float32),
            pltpu.SemaphoreType.DMA,
            pltpu.SemaphoreType.DMA,
            pltpu.SemaphoreType.DMA,
            pltpu.SemaphoreType.DMA,
        ],
    )(x3, w_flat)
    return jnp.transpose(out_lin.reshape(_COLS, 1, _ROWS), (2, 0, 1))


# speedup vs baseline: 1.6768x; 1.6768x over previous
import jax
import jax.numpy as jnp
from jax import lax
from jax.experimental import pallas as pl
from jax.experimental.pallas import tpu as pltpu

_ROWS = 16384
_COLS = 200
_RB = 8
_GRID = _COLS // _RB
_NBUF = 6


def _tc_body(w_ref, x_ref, out_ref, acc_ref, sem):
    i = pl.program_id(0)
    b = lax.rem(i, _NBUF)

    def dma(step, buf, r):
        return pltpu.make_async_copy(
            acc_ref.at[buf, r], out_ref.at[step * _RB + r, 0], sem)

    @pl.when(i >= _NBUF - 1)
    def _():
        j = i - (_NBUF - 1)
        for r in range(_RB):
            dma(j, lax.rem(j, _NBUF), r).wait()

    xb = x_ref[...]
    w0 = w_ref[0, 0]
    w1 = w_ref[0, 1]
    w2 = w_ref[0, 2]
    w3 = w_ref[0, 3]
    lo = jnp.where(xb == 1, w1, w0)
    hi = jnp.where(xb == 3, w3, w2)
    acc_ref[b] = jnp.where(xb >= 2, hi, lo)
    for r in range(_RB):
        dma(i, b, r).start()

    @pl.when(i == _GRID - 1)
    def _():
        for j in range(_GRID - (_NBUF - 1), _GRID):
            for r in range(_RB):
                dma(j, j % _NBUF, r).wait()


@jax.jit
def kernel(x, weight):
    w_row = weight.reshape(1, 4).astype(jnp.float32)
    xt = x.T
    out_lin = pl.pallas_call(
        _tc_body,
        grid=(_GRID,),
        in_specs=[
            pl.BlockSpec(memory_space=pltpu.SMEM),
            pl.BlockSpec((_RB, _ROWS), lambda i: (i, 0)),
        ],
        out_specs=pl.BlockSpec(memory_space=pl.ANY),
        out_shape=jax.ShapeDtypeStruct((_COLS, 1, _ROWS), jnp.float32),
        scratch_shapes=[
            pltpu.VMEM((_NBUF, _RB, _ROWS), jnp.float32),
            pltpu.SemaphoreType.DMA,
        ],
    )(w_row, xt)
    return jnp.transpose(out_lin, (2, 0, 1))
